# baseline (speedup 1.0000x reference)
import jax
import jax.numpy as jnp
from jax import lax
from jax.experimental import pallas as pl
from jax.experimental.pallas import tpu as pltpu

B, D = 16384, 128
CH = 1024
NCH = B // CH
RANK = D // 2


def _compute(lhs, rel, first):
    rrel = pltpu.roll(rel, RANK, 1)
    rlhs = pltpu.roll(lhs, RANK, 1)
    a = jnp.where(first, rel, rrel)
    b = jnp.where(first, -rrel, rel)
    return lhs * a + rlhs * b


def _body(lhs_hbm, rel_hbm, out_hbm, lv, rv, ov, sl, sr, so):
    col = lax.broadcasted_iota(jnp.int32, (CH, D), 1)
    first = col < RANK

    def in_copy(ci, slot):
        row0 = ci * CH
        cl = pltpu.make_async_copy(lhs_hbm.at[pl.ds(row0, CH)], lv.at[slot], sl.at[slot])
        cr = pltpu.make_async_copy(rel_hbm.at[pl.ds(row0, CH)], rv.at[slot], sr.at[slot])
        return cl, cr

    def out_copy(ci, slot):
        row0 = ci * CH
        return pltpu.make_async_copy(ov.at[slot], out_hbm.at[pl.ds(row0, CH)], so.at[slot])

    cl0, cr0 = in_copy(0, 0)
    cl0.start()
    cr0.start()

    def step(ci, carry):
        slot = lax.rem(ci, 2)
        nslot = lax.rem(ci + 1, 2)

        @pl.when(ci + 1 < NCH)
        def _():
            ncl, ncr = in_copy(ci + 1, nslot)
            ncl.start()
            ncr.start()

        cl, cr = in_copy(ci, slot)
        cl.wait()
        cr.wait()

        @pl.when(ci >= 2)
        def _():
            out_copy(ci - 2, slot).wait()

        ov[slot] = _compute(lv[slot], rv[slot], first)
        out_copy(ci, slot).start()
        return carry

    lax.fori_loop(0, NCH, step, 0)
    out_copy(NCH - 2, lax.rem(NCH - 2, 2)).wait()
    out_copy(NCH - 1, lax.rem(NCH - 1, 2)).wait()


def kernel(lhs, rel):
    return pl.pallas_call(
        _body,
        in_specs=[
            pl.BlockSpec(memory_space=pltpu.HBM),
            pl.BlockSpec(memory_space=pltpu.HBM),
        ],
        out_specs=pl.BlockSpec(memory_space=pltpu.HBM),
        out_shape=jax.ShapeDtypeStruct((B, D), lhs.dtype),
        scratch_shapes=[
            pltpu.VMEM((2, CH, D), jnp.float32),
            pltpu.VMEM((2, CH, D), jnp.float32),
            pltpu.VMEM((2, CH, D), jnp.float32),
            pltpu.SemaphoreType.DMA((2,)),
            pltpu.SemaphoreType.DMA((2,)),
            pltpu.SemaphoreType.DMA((2,)),
        ],
    )(lhs, rel)


# speedup vs baseline: 1.0529x; 1.0529x over previous
import jax
import jax.numpy as jnp
from jax import lax
from jax.experimental import pallas as pl
from jax.experimental.pallas import tpu as pltpu

B, D = 16384, 128
CH = 1024
NCH = B // CH
RANK = D // 2


def _compute(lhs, rel, first):
    rrel = pltpu.roll(rel, RANK, 1)
    rlhs = pltpu.roll(lhs, RANK, 1)
    a = jnp.where(first, rel, rrel)
    b = jnp.where(first, -rrel, rel)
    return lhs * a + rlhs * b


def _body(lhs_hbm, rel_hbm, out_hbm, lv, rv, ov, sl, sr, so):
    col = lax.broadcasted_iota(jnp.int32, (CH, D), 1)
    first = col < RANK

    def in_copy(ci, slot):
        row0 = ci * CH
        cl = pltpu.make_async_copy(lhs_hbm.at[pl.ds(row0, CH)], lv.at[slot], sl.at[slot])
        cr = pltpu.make_async_copy(rel_hbm.at[pl.ds(row0, CH)], rv.at[slot], sr.at[slot])
        return cl, cr

    def out_copy(ci, slot):
        row0 = ci * CH
        return pltpu.make_async_copy(ov.at[slot], out_hbm.at[pl.ds(row0, CH)], so.at[slot])

    cl0, cr0 = in_copy(0, 0)
    cl0.start()
    cr0.start()

    pend_in = {0: (cl0, cr0)}
    pend_out = {}
    for ci in range(NCH):
        slot = ci % 2
        if ci + 1 < NCH:
            nc = in_copy(ci + 1, (ci + 1) % 2)
            nc[0].start()
            nc[1].start()
            pend_in[ci + 1] = nc
        cl, cr = pend_in.pop(ci)
        cl.wait()
        cr.wait()
        if ci - 2 in pend_out:
            pend_out.pop(ci - 2).wait()
        ov[slot] = _compute(lv[slot], rv[slot], first)
        co = out_copy(ci, slot)
        co.start()
        pend_out[ci] = co
    for co in pend_out.values():
        co.wait()


def kernel(lhs, rel):
    return pl.pallas_call(
        _body,
        in_specs=[
            pl.BlockSpec(memory_space=pltpu.HBM),
            pl.BlockSpec(memory_space=pltpu.HBM),
        ],
        out_specs=pl.BlockSpec(memory_space=pltpu.HBM),
        out_shape=jax.ShapeDtypeStruct((B, D), lhs.dtype),
        scratch_shapes=[
            pltpu.VMEM((2, CH, D), jnp.float32),
            pltpu.VMEM((2, CH, D), jnp.float32),
            pltpu.VMEM((2, CH, D), jnp.float32),
            pltpu.SemaphoreType.DMA((2,)),
            pltpu.SemaphoreType.DMA((2,)),
            pltpu.SemaphoreType.DMA((2,)),
        ],
    )(lhs, rel)


# speedup vs baseline: 1.6493x; 1.5664x over previous
import jax
import jax.numpy as jnp
from jax import lax
from jax.experimental import pallas as pl
from jax.experimental.pallas import tpu as pltpu


def _complex_body(lhs_ref, rel_ref, out_ref):
    lhs = lhs_ref[...]
    rel = rel_ref[...]
    n, d = lhs.shape
    r = d // 2
    col = lax.broadcasted_iota(jnp.int32, (n, d), 1)
    first = col < r
    rrel = pltpu.roll(rel, r, 1)
    rlhs = pltpu.roll(lhs, r, 1)
    a = jnp.where(first, rel, rrel)
    b = jnp.where(first, -rrel, rel)
    out_ref[...] = lhs * a + rlhs * b


def kernel(lhs, rel):
    B, D = lhs.shape
    blk = 8192
    return pl.pallas_call(
        _complex_body,
        grid=(B // blk,),
        in_specs=[
            pl.BlockSpec((blk, D), lambda i: (i, 0)),
            pl.BlockSpec((blk, D), lambda i: (i, 0)),
        ],
        out_specs=pl.BlockSpec((blk, D), lambda i: (i, 0)),
        out_shape=jax.ShapeDtypeStruct((B, D), lhs.dtype),
    )(lhs, rel)
